# baseline (speedup 1.0000x reference)
import functools

import jax
import jax.numpy as jnp
import numpy as np
from jax.experimental import pallas as pl
from jax.experimental.shard_map import shard_map
from jax.sharding import Mesh, PartitionSpec as P

N = 4096
D = 128
O = 32
BR = 512


def _deg_proj_kernel(adj_ref, adjs_ref, adjt_ref, inps_ref, inpt_ref, w_ref,
                     dso_ref, dto_ref, x_ref, y_ref):
    i = pl.program_id(0)
    a = adj_ref[...]
    dso_ref[...] = (jnp.sum(a, axis=1, keepdims=True)
                    + jnp.sum(adjs_ref[...], axis=1, keepdims=True))
    csum = (jnp.sum(a, axis=0, keepdims=True)
            + jnp.sum(adjt_ref[...], axis=0, keepdims=True))

    @pl.when(i == 0)
    def _():
        dto_ref[...] = csum

    @pl.when(i > 0)
    def _():
        dto_ref[...] += csum

    x_ref[...] = jnp.dot(inps_ref[...], w_ref[...],
                         preferred_element_type=jnp.float32)
    y_ref[...] = jnp.dot(inpt_ref[...], w_ref[...],
                         preferred_element_type=jnp.float32)


def _xnew_kernel(adj_ref, adjs_ref, x_ref, y_ref, xloc_ref, dso_ref,
                 dsoloc_ref, dtoc_ref, xn_ref, yp_ref):
    i = pl.program_id(0)
    rds_full = jnp.sqrt(dso_ref[...] + 1.0)
    rdt_full = jnp.sqrt(dtoc_ref[...] + 1.0)
    xs = (x_ref[...] / rds_full).astype(jnp.bfloat16)
    yt = (y_ref[...] / rdt_full).astype(jnp.bfloat16)
    a = adj_ref[...].astype(jnp.bfloat16)
    acc = (jnp.dot(adjs_ref[...].astype(jnp.bfloat16), xs,
                   preferred_element_type=jnp.float32)
           + jnp.dot(a, yt, preferred_element_type=jnp.float32))
    x_blk = xloc_ref[pl.ds(i * BR, BR), :]
    rds_blk = jnp.sqrt(dsoloc_ref[pl.ds(i * BR, BR), :] + 1.0)
    xn = jnp.maximum((x_blk + acc) / rds_blk, 0.0)
    xn_ref[...] = xn
    contrib = jax.lax.dot_general(a, (xn / rds_blk).astype(jnp.bfloat16),
                                  (((0,), (0,)), ((), ())),
                                  preferred_element_type=jnp.float32)

    @pl.when(i == 0)
    def _():
        yp_ref[...] = contrib

    @pl.when(i > 0)
    def _():
        yp_ref[...] += contrib


def _ynew_kernel(adjt_ref, y_ref, yloc_ref, dtoc_ref, dtocloc_ref, yp_ref,
                 yn_ref):
    i = pl.program_id(0)
    rdt_full = jnp.sqrt(dtoc_ref[...] + 1.0)
    yt = (y_ref[...] / rdt_full).astype(jnp.bfloat16)
    acc = jnp.dot(adjt_ref[...].astype(jnp.bfloat16), yt,
                  preferred_element_type=jnp.float32)
    y_blk = yloc_ref[pl.ds(i * BR, BR), :]
    yp_blk = yp_ref[pl.ds(i * BR, BR), :]
    rdt_blk = jnp.sqrt(dtocloc_ref[pl.ds(i * BR, BR), :] + 1.0)
    yn_ref[...] = jnp.maximum((y_blk + acc + yp_blk) / rdt_blk, 0.0)


def _shard_body(inp_s, inp_t, adj_c, adjs_c, adjt_c, W):
    R = adj_c.shape[0]
    nb = R // BR
    bp = N // nb

    row_blk = pl.BlockSpec((BR, N), lambda i: (i, 0))
    full = lambda shape: pl.BlockSpec(shape, lambda i: (0, 0))

    dso_c, dto_part, x, y = pl.pallas_call(
        _deg_proj_kernel,
        grid=(nb,),
        in_specs=[row_blk, row_blk, row_blk,
                  pl.BlockSpec((bp, D), lambda i: (i, 0)),
                  pl.BlockSpec((bp, D), lambda i: (i, 0)),
                  full((D, O))],
        out_specs=[pl.BlockSpec((BR, 1), lambda i: (i, 0)),
                   full((1, N)),
                   pl.BlockSpec((bp, O), lambda i: (i, 0)),
                   pl.BlockSpec((bp, O), lambda i: (i, 0))],
        out_shape=[jax.ShapeDtypeStruct((R, 1), jnp.float32),
                   jax.ShapeDtypeStruct((1, N), jnp.float32),
                   jax.ShapeDtypeStruct((N, O), jnp.float32),
                   jax.ShapeDtypeStruct((N, O), jnp.float32)],
    )(adj_c, adjs_c, adjt_c, inp_s, inp_t, W)

    dto = jax.lax.psum(dto_part, "c")
    dso = jax.lax.all_gather(dso_c, "c", axis=0, tiled=True)
    dto_col = dto.reshape(N, 1)

    ci = jax.lax.axis_index("c")
    x_c = jax.lax.dynamic_slice(x, (ci * R, 0), (R, O))
    y_c = jax.lax.dynamic_slice(y, (ci * R, 0), (R, O))
    dtoc_c = jax.lax.dynamic_slice(dto_col, (ci * R, 0), (R, 1))

    xn_c, yp_part = pl.pallas_call(
        _xnew_kernel,
        grid=(nb,),
        in_specs=[row_blk, row_blk, full((N, O)), full((N, O)),
                  full((R, O)), full((N, 1)), full((R, 1)), full((N, 1))],
        out_specs=[pl.BlockSpec((BR, O), lambda i: (i, 0)),
                   full((N, O))],
        out_shape=[jax.ShapeDtypeStruct((R, O), jnp.float32),
                   jax.ShapeDtypeStruct((N, O), jnp.float32)],
    )(adj_c, adjs_c, x, y, x_c, dso, dso_c, dto_col)

    yp_c = jax.lax.psum_scatter(yp_part, "c", scatter_dimension=0,
                                tiled=True)

    yn_c = pl.pallas_call(
        _ynew_kernel,
        grid=(nb,),
        in_specs=[row_blk, full((N, O)), full((R, O)), full((N, 1)),
                  full((R, 1)), full((R, O))],
        out_specs=pl.BlockSpec((BR, O), lambda i: (i, 0)),
        out_shape=jax.ShapeDtypeStruct((R, O), jnp.float32),
    )(adjt_c, y, y_c, dto_col, dtoc_c, yp_c)

    return xn_c, yn_c


def kernel(inp_s, inp_t, adj, adj_s, adj_t, W):
    devs = [d for d in jax.devices() if d.platform == "tpu"] or jax.devices()
    C = 2 if len(devs) >= 2 else 1
    mesh = Mesh(np.array(devs[:C]), ("c",))
    f = shard_map(
        _shard_body, mesh=mesh,
        in_specs=(P(None, None), P(None, None), P("c", None), P("c", None),
                  P("c", None), P(None, None)),
        out_specs=(P("c", None), P("c", None)),
        check_rep=False,
    )
    return f(inp_s, inp_t, adj, adj_s, adj_t, W)


# speedup vs baseline: 3.8302x; 3.8302x over previous
import jax
import jax.numpy as jnp
from jax.experimental import pallas as pl
from jax.experimental.pallas import tpu as pltpu

N = 4096
D = 128
O = 32
BR = 128
NB = N // BR


def _fused_kernel(adj_ref, adjs_ref, adjt_ref, inps_ref, inpt_ref, w_ref,
                  xn_ref, yn_ref,
                  stash_ref,
                  dso_ref,
                  dto_ref,
                  x_ref,
                  y_ref,
                  rds_ref,
                  rdt_ref,
                  xs_ref,
                  yt_ref,
                  yp_ref):
    i = pl.program_id(0)

    @pl.when(i < NB)
    def _phase1():
        a = adj_ref[...]
        asrc = adjs_ref[...]
        stash_ref[pl.ds(i * BR, BR), :] = asrc.astype(jnp.bfloat16)
        dso_ref[pl.ds(i * BR, BR), :] = (
            jnp.sum(a, axis=1, keepdims=True)
            + jnp.sum(asrc, axis=1, keepdims=True))
        csum = (jnp.sum(a, axis=0, keepdims=True)
                + jnp.sum(adjt_ref[...], axis=0, keepdims=True))

        @pl.when(i == 0)
        def _():
            dto_ref[...] = csum

        @pl.when(i > 0)
        def _():
            dto_ref[...] += csum

        x_ref[pl.ds(i * BR, BR), :] = jnp.dot(
            inps_ref[...], w_ref[...], preferred_element_type=jnp.float32)
        y_ref[pl.ds(i * BR, BR), :] = jnp.dot(
            inpt_ref[...], w_ref[...], preferred_element_type=jnp.float32)

    @pl.when(i == NB)
    def _precompute():
        rds = jnp.sqrt(dso_ref[...] + 1.0)
        rdt = jnp.sqrt(dto_ref[...].reshape(N, 1) + 1.0)
        rds_ref[...] = rds
        rdt_ref[...] = rdt
        xs_ref[...] = (x_ref[...] / rds).astype(jnp.bfloat16)
        yt_ref[...] = (y_ref[...] / rdt).astype(jnp.bfloat16)

    @pl.when(jnp.logical_and(i >= NB, i < 2 * NB))
    def _phase2():
        j = i - NB
        a = adj_ref[...].astype(jnp.bfloat16)
        a_s = stash_ref[pl.ds(j * BR, BR), :]
        acc = (jnp.dot(a_s, xs_ref[...], preferred_element_type=jnp.float32)
               + jnp.dot(a, yt_ref[...], preferred_element_type=jnp.float32))
        x_blk = x_ref[pl.ds(j * BR, BR), :]
        rds_blk = rds_ref[pl.ds(j * BR, BR), :]
        xn = jnp.maximum((x_blk + acc) / rds_blk, 0.0)
        xn_ref[...] = xn
        contrib = jax.lax.dot_general(
            a, (xn / rds_blk).astype(jnp.bfloat16),
            (((0,), (0,)), ((), ())), preferred_element_type=jnp.float32)

        @pl.when(j == 0)
        def _():
            yp_ref[...] = contrib

        @pl.when(j > 0)
        def _():
            yp_ref[...] += contrib

    @pl.when(i >= 2 * NB)
    def _phase3():
        k = i - 2 * NB
        at = adjt_ref[...].astype(jnp.bfloat16)
        acc = jnp.dot(at, yt_ref[...], preferred_element_type=jnp.float32)
        y_blk = y_ref[pl.ds(k * BR, BR), :]
        yp_blk = yp_ref[pl.ds(k * BR, BR), :]
        rdt_blk = rdt_ref[pl.ds(k * BR, BR), :]
        yn_ref[...] = jnp.maximum((y_blk + acc + yp_blk) / rdt_blk, 0.0)


def kernel(inp_s, inp_t, adj, adj_s, adj_t, W):
    last = NB - 1
    adj_map = lambda i: (jnp.where(i < 2 * NB, i % NB, last), 0)
    adjs_map = lambda i: (jnp.where(i < NB, i, last), 0)
    adjt_map = lambda i: (
        jnp.where(i < NB, i, jnp.where(i < 2 * NB, last, i - 2 * NB)), 0)
    inp_map = lambda i: (jnp.where(i < NB, i, last), 0)
    xn_map = lambda i: (jnp.where(i < NB, 0, jnp.where(i < 2 * NB, i - NB, last)), 0)
    yn_map = lambda i: (jnp.where(i < 2 * NB, 0, i - 2 * NB), 0)

    xn, yn = pl.pallas_call(
        _fused_kernel,
        grid=(3 * NB,),
        in_specs=[pl.BlockSpec((BR, N), adj_map),
                  pl.BlockSpec((BR, N), adjs_map),
                  pl.BlockSpec((BR, N), adjt_map),
                  pl.BlockSpec((BR, D), inp_map),
                  pl.BlockSpec((BR, D), inp_map),
                  pl.BlockSpec((D, O), lambda i: (0, 0))],
        out_specs=[pl.BlockSpec((BR, O), xn_map),
                   pl.BlockSpec((BR, O), yn_map)],
        out_shape=[jax.ShapeDtypeStruct((N, O), jnp.float32),
                   jax.ShapeDtypeStruct((N, O), jnp.float32)],
        scratch_shapes=[pltpu.VMEM((N, N), jnp.bfloat16),
                        pltpu.VMEM((N, 1), jnp.float32),
                        pltpu.VMEM((1, N), jnp.float32),
                        pltpu.VMEM((N, O), jnp.float32),
                        pltpu.VMEM((N, O), jnp.float32),
                        pltpu.VMEM((N, 1), jnp.float32),
                        pltpu.VMEM((N, 1), jnp.float32),
                        pltpu.VMEM((N, O), jnp.bfloat16),
                        pltpu.VMEM((N, O), jnp.bfloat16),
                        pltpu.VMEM((N, O), jnp.float32)],
        compiler_params=pltpu.CompilerParams(
            vmem_limit_bytes=128 * 1024 * 1024),
    )(adj, adj_s, adj_t, inp_s, inp_t, W)

    return (xn, yn)
